# baseline (speedup 1.0000x reference)
import functools

import jax
import jax.numpy as jnp
from jax import lax
from jax.experimental import pallas as pl
from jax.experimental.pallas import tpu as pltpu
from jax.experimental.pallas import tpu_sc as plsc

VOCAB = 100000
EMB = 64
SCALE = 8.0

NC = 2
NS = 16
NW = NC * NS

BLK = 128
PITCH = BLK + 1
NBUF = 2


def _make_sc_kernel(n_l, n_b):
    toks_per_w = (n_b // NW) * n_l
    idx_src_pad = toks_per_w + 16
    idx_dst_pad = (n_l + 16) * BLK

    mesh = plsc.VectorSubcoreMesh(core_axis_name="c", subcore_axis_name="s")

    @functools.partial(
        pl.kernel,
        mesh=mesh,
        compiler_params=pltpu.CompilerParams(use_tc_tiling_on_sc=False,
                                             needs_layout_passes=False),
        out_type=jax.ShapeDtypeStruct((n_l, EMB // 8, n_b // BLK, 8, BLK),
                                      jnp.float32),
        scratch_types=[
            pltpu.VMEM((idx_src_pad,), jnp.int32),
            pltpu.VMEM((idx_dst_pad,), jnp.int32),
            pltpu.VMEM((BLK, EMB), jnp.float32),
            pltpu.VMEM((BLK, EMB), jnp.float32),
            pltpu.VMEM((EMB, PITCH), jnp.float32),
            pltpu.VMEM((EMB, PITCH), jnp.float32),
            pltpu.SemaphoreType.DMA,
            pltpu.SemaphoreType.DMA,
            pltpu.SemaphoreType.DMA,
            pltpu.SemaphoreType.DMA,
        ],
    )
    def sc_kernel(tok_hbm, tab_hbm, out_hbm, idx_bm, idx_lm,
                  rows0, rows1, tr0, tr1, semg0, semg1, semw0, semw1):
        wid = lax.axis_index("s") * NC + lax.axis_index("c")

        pltpu.sync_copy(tok_hbm.at[pl.ds(wid * toks_per_w, toks_per_w)],
                        idx_bm.at[pl.ds(0, toks_per_w)])

        rows = (rows0, rows1)
        trs = (tr0, tr1)
        semg = (semg0, semg1)
        semw = (semw0, semw1)

        iota = lax.iota(jnp.int32, 16)
        perms = [(iota + d) & 15 for d in range(16)]
        rowv = [iota + 16 * m for m in range(EMB // 16)]
        srcv = [iota * n_l + p for p in perms]
        dstv = [p * BLK + iota for p in perms]

        l0s = list(range(0, n_l - 8, 16)) + [n_l - 16]

        @plsc.parallel_loop(0, BLK, step=16)
        def idx_transpose(b0):
            for l0 in l0s:
                for d in range(16):
                    v = plsc.load_gather(idx_bm, [srcv[d] + (b0 * n_l + l0)])
                    plsc.store_scatter(idx_lm, [dstv[d] + (l0 * BLK + b0)], v)

        def fire_gather(i, j):
            pltpu.async_copy(
                tab_hbm.at[idx_lm.at[pl.ds(i * BLK, BLK)]], rows[j], semg[j])

        def drain_gather(j):
            pltpu.make_async_copy(tab_hbm.at[pl.ds(0, BLK)], rows[j],
                                  semg[j]).wait()

        def fire_write(i, j):
            for et in range(EMB // 8):
                pltpu.async_copy(trs[j].at[pl.ds(8 * et, 8), pl.ds(0, BLK)],
                                 out_hbm.at[i, et, wid], semw[j])

        def drain_write(j):
            for et in range(EMB // 8):
                pltpu.make_async_copy(trs[j].at[pl.ds(8 * et, 8),
                                                pl.ds(0, BLK)],
                                      out_hbm.at[0, 0, 0], semw[j]).wait()

        def transpose_block(rbuf, tbuf):
            @plsc.parallel_loop(0, BLK, step=8, unroll=2)
            def body(bi0):
                for k in range(8):
                    bi = bi0 + k
                    bv = jnp.zeros((16,), jnp.int32) + bi
                    for m in range(EMB // 16):
                        v = rbuf[bi, pl.ds(16 * m, 16)] * SCALE
                        plsc.store_scatter(tbuf, [rowv[m], bv], v)

        def process(i, j, first, last):
            if not first:
                drain_write(j)
            drain_gather(j)
            transpose_block(rows[j], trs[j])
            if not last:
                fire_gather(i + NBUF, j)
            fire_write(i, j)

        for j in range(NBUF):
            fire_gather(j, j)

        def outer(c, carry):
            i0 = c * NBUF
            for j in range(NBUF):
                process(i0 + j, j, first=False, last=False)
            return carry

        for j in range(NBUF):
            process(j, j, first=True, last=False)
        lax.fori_loop(1, n_l // NBUF - 1, outer, 0)
        for j in range(NBUF):
            process(n_l - NBUF + j, j, first=False, last=True)
        for j in range(NBUF):
            drain_write(j)

    return sc_kernel


@jax.jit
def _embed(tokens, table):
    B, L = tokens.shape
    tok_flat = tokens.reshape(-1).astype(jnp.int32)
    out5 = _make_sc_kernel(L, B)(tok_flat, table)
    out = jnp.transpose(out5, (0, 1, 3, 2, 4)).reshape(L, EMB, B)
    return jnp.transpose(out, (2, 0, 1))


def kernel(tokens, table):
    return _embed(tokens, table)


# speedup vs baseline: 1.3643x; 1.3643x over previous
import functools

import jax
import jax.numpy as jnp
from jax import lax
from jax.experimental import pallas as pl
from jax.experimental.pallas import tpu as pltpu
from jax.experimental.pallas import tpu_sc as plsc

VOCAB = 100000
EMB = 64
SCALE = 8.0

NC = 2
NS = 16
NW = NC * NS

BLK = 128
PITCH = BLK + 1
NBUF = 2


def _make_sc_kernel(n_l, n_b):
    toks_per_w = (n_b // NW) * n_l
    idx_src_pad = toks_per_w + 16
    idx_dst_pad = (n_l + 16) * BLK

    mesh = plsc.VectorSubcoreMesh(core_axis_name="c", subcore_axis_name="s")

    @functools.partial(
        pl.kernel,
        mesh=mesh,
        compiler_params=pltpu.CompilerParams(use_tc_tiling_on_sc=False,
                                             needs_layout_passes=False),
        out_type=jax.ShapeDtypeStruct((n_l, EMB // 8, n_b // BLK, 8, BLK),
                                      jnp.float32),
        scratch_types=[
            pltpu.VMEM((idx_src_pad,), jnp.int32),
            pltpu.VMEM((idx_dst_pad,), jnp.int32),
            pltpu.VMEM((BLK, EMB), jnp.float32),
            pltpu.VMEM((BLK, EMB), jnp.float32),
            pltpu.VMEM((EMB, PITCH), jnp.float32),
            pltpu.VMEM((EMB, PITCH), jnp.float32),
            pltpu.SemaphoreType.DMA,
            pltpu.SemaphoreType.DMA,
            pltpu.SemaphoreType.DMA,
            pltpu.SemaphoreType.DMA,
        ],
    )
    def sc_kernel(tok_hbm, tab_hbm, out_hbm, idx_bm, idx_lm,
                  rows0, rows1, tr0, tr1, semg0, semg1, semw0, semw1):
        wid = lax.axis_index("s") * NC + lax.axis_index("c")

        pltpu.sync_copy(tok_hbm.at[pl.ds(wid * toks_per_w, toks_per_w)],
                        idx_bm.at[pl.ds(0, toks_per_w)])

        rows = (rows0, rows1)
        trs = (tr0, tr1)
        semg = (semg0, semg1)
        semw = (semw0, semw1)

        iota = lax.iota(jnp.int32, 16)
        perms = [(iota + d) & 15 for d in range(16)]
        rowv = [iota + 16 * m for m in range(EMB // 16)]
        srcv = [iota * n_l + p for p in perms]
        dstv = [p * BLK + iota for p in perms]

        l0s = list(range(0, n_l - 8, 16)) + [n_l - 16]

        @plsc.parallel_loop(0, BLK, step=16)
        def idx_transpose(b0):
            for l0 in l0s:
                for d in range(16):
                    v = plsc.load_gather(idx_bm, [srcv[d] + (b0 * n_l + l0)])
                    plsc.store_scatter(idx_lm, [dstv[d] + (l0 * BLK + b0)], v)

        def fire_gather(i, j):
            pltpu.async_copy(
                tab_hbm.at[idx_lm.at[pl.ds(i * BLK, BLK)]], rows[j], semg[j])

        def drain_gather(j):
            pltpu.make_async_copy(tab_hbm.at[pl.ds(0, BLK)], rows[j],
                                  semg[j]).wait()

        def fire_write(i, j):
            for et in range(EMB // 8):
                pltpu.async_copy(trs[j].at[pl.ds(8 * et, 8), pl.ds(0, BLK)],
                                 out_hbm.at[i, et, wid], semw[j])

        def drain_write(j):
            for et in range(EMB // 8):
                pltpu.make_async_copy(trs[j].at[pl.ds(8 * et, 8),
                                                pl.ds(0, BLK)],
                                      out_hbm.at[0, 0, 0], semw[j]).wait()

        def transpose_block(rbuf, tbuf):
            @plsc.parallel_loop(0, BLK, step=4, unroll=4)
            def body(bi0):
                for k in range(4):
                    bi = bi0 + k
                    bv = jnp.zeros((16,), jnp.int32) + bi
                    for m in range(EMB // 16):
                        v = rbuf[bi, pl.ds(16 * m, 16)] * SCALE
                        plsc.store_scatter(tbuf, [rowv[m], bv], v)

        def process(i, j, first, last):
            if not first:
                drain_write(j)
            drain_gather(j)
            transpose_block(rows[j], trs[j])
            if not last:
                fire_gather(i + NBUF, j)
            fire_write(i, j)

        for j in range(NBUF):
            fire_gather(j, j)

        def outer(c, carry):
            i0 = c * NBUF
            for j in range(NBUF):
                process(i0 + j, j, first=False, last=False)
            return carry

        for j in range(NBUF):
            process(j, j, first=True, last=False)
        lax.fori_loop(1, n_l // NBUF - 1, outer, 0)
        for j in range(NBUF):
            process(n_l - NBUF + j, j, first=False, last=True)
        for j in range(NBUF):
            drain_write(j)

    return sc_kernel


@jax.jit
def _embed(tokens, table):
    B, L = tokens.shape
    tok_flat = tokens.reshape(-1).astype(jnp.int32)
    out5 = _make_sc_kernel(L, B)(tok_flat, table)
    out = jnp.transpose(out5, (0, 1, 3, 2, 4)).reshape(L, EMB, B)
    return jnp.transpose(out, (2, 0, 1))


def kernel(tokens, table):
    return _embed(tokens, table)
